# Initial kernel scaffold; baseline (speedup 1.0000x reference)
import jax
import jax.numpy as jnp
from jax.experimental import pallas as pl


def kernel(x, edge_index, W, b):
    raise NotImplementedError("write your pallas kernel here")


# speedup vs baseline: 4.2706x; 4.2706x over previous
import functools

import jax
import jax.numpy as jnp
import numpy as np
from jax import lax
from jax.experimental import pallas as pl
from jax.experimental.pallas import tpu as pltpu
from jax.experimental.pallas import tpu_sc as plsc

N = 10000
F = 128
FEAT_EMB = 15
VAL_EMB = 1
EMB = FEAT_EMB + VAL_EMB
CHANNELS = F * EMB
OUT = 7
E = 640000

OUTP = 8
PADN = 10240
NC = 2
NS = 16
NT = NC * NS
CW = 128
TCH = 160
EPAD = NT * TCH * CW
SL = PADN // NS

@functools.cache
def _mesh():
    return plsc.VectorSubcoreMesh(
        core_axis_name="c", subcore_axis_name="s", num_cores=NC, num_subcores=NS
    )


def _embed_like(x):
    Xt = x.T
    Xc = Xt - Xt.mean(axis=0, keepdims=True)
    U, S, Vt = jnp.linalg.svd(Xc, full_matrices=False)
    feat_emb = U[:, :FEAT_EMB] * S[:FEAT_EMB]
    vals = x.reshape(N * F, 1)
    vals = jnp.repeat(vals, VAL_EMB, axis=1)
    fe = jnp.tile(feat_emb, (N, 1))
    cat = jnp.concatenate([fe, vals], axis=1)
    node_vec = cat.reshape(N, F * EMB)
    mu = node_vec.mean(axis=0, keepdims=True)
    sd = node_vec.std(axis=0, keepdims=True)
    return (node_vec - mu) / (sd + 1e-12)


def _deg_body(dst_hbm, z_hbm, ones_hbm, deg_out, idx_v, ones_v, deg_sh, sem):
    del sem
    c = lax.axis_index("c")
    s = lax.axis_index("s")
    tid = s * NC + c
    pltpu.sync_copy(z_hbm.at[pl.ds(s * SL, SL)], deg_sh.at[pl.ds(s * SL, SL)])
    pltpu.sync_copy(ones_hbm, ones_v)
    pltpu.sync_copy(dst_hbm.at[tid], idx_v)
    plsc.subcore_barrier()

    def step(j, carry):
        pltpu.sync_copy(ones_v, deg_sh.at[idx_v.at[j]], add=True)
        return carry

    lax.fori_loop(0, TCH, step, 0)
    plsc.subcore_barrier()

    @pl.when(s == 0)
    def _():
        pltpu.sync_copy(deg_sh, deg_out.at[c])


@functools.cache
def _sc_deg():
    return pl.kernel(
        _deg_body,
        out_type=jax.ShapeDtypeStruct((NC, PADN), jnp.float32),
        mesh=_mesh(),
        scratch_types=[
            pltpu.VMEM((TCH, CW), jnp.int32),
            pltpu.VMEM((CW,), jnp.float32),
            pltpu.VMEM_SHARED((PADN,), jnp.float32),
            pltpu.SemaphoreType.DMA,
        ],
    )


def _hw_body(h_ref, wp_ref, degt_ref, hws_ref, dinv_ref):
    deg = degt_ref[:, 0] + degt_ref[:, 1] + 1.0
    dinv = lax.rsqrt(deg)
    hw = jnp.dot(h_ref[...], wp_ref[...], preferred_element_type=jnp.float32)
    hws_ref[...] = hw * dinv[:, None]
    dinv_ref[...] = dinv[:, None]


_BL = 400


def _tc_hw(h, wp, degt):
    return pl.pallas_call(
        _hw_body,
        grid=(N // _BL,),
        in_specs=[
            pl.BlockSpec((_BL, CHANNELS), lambda i: (i, 0)),
            pl.BlockSpec((CHANNELS, OUTP), lambda i: (0, 0)),
            pl.BlockSpec((_BL, 2), lambda i: (i, 0)),
        ],
        out_specs=[
            pl.BlockSpec((_BL, OUTP), lambda i: (i, 0)),
            pl.BlockSpec((_BL, 1), lambda i: (i, 0)),
        ],
        out_shape=[
            jax.ShapeDtypeStruct((N, OUTP), jnp.float32),
            jax.ShapeDtypeStruct((N, 1), jnp.float32),
        ],
    )(h, wp, degt)


_FLATN = OUTP * PADN
_SL8 = _FLATN // NS


def _edges_body(src_hbm, dst_hbm, hwst_hbm, z8_hbm, acc_out,
                sidx, didx, gidx, widx, rowbuf, acc_sh, sem):
    del sem
    c = lax.axis_index("c")
    s = lax.axis_index("s")
    tid = s * NC + c
    pltpu.sync_copy(z8_hbm.at[pl.ds(s * _SL8, _SL8)],
                    acc_sh.at[pl.ds(s * _SL8, _SL8)])
    pltpu.sync_copy(src_hbm.at[tid], sidx)
    pltpu.sync_copy(dst_hbm.at[tid], didx)
    plsc.subcore_barrier()

    def step(j, carry):
        for p in range(OUTP - 1):
            for k in range(CW // 16):
                sl = pl.ds(k * 16, 16)
                gidx[sl] = sidx[j, sl] + p * PADN
                widx[sl] = didx[j, sl] + p * PADN
            pltpu.sync_copy(hwst_hbm.at[gidx], rowbuf)
            pltpu.sync_copy(rowbuf, acc_sh.at[widx], add=True)
        return carry

    lax.fori_loop(0, TCH, step, 0)
    plsc.subcore_barrier()

    @pl.when(s == 0)
    def _():
        pltpu.sync_copy(acc_sh, acc_out.at[c])


@functools.cache
def _sc_edges():
    return pl.kernel(
        _edges_body,
        out_type=jax.ShapeDtypeStruct((NC, _FLATN), jnp.float32),
        mesh=_mesh(),
        scratch_types=[
            pltpu.VMEM((TCH, CW), jnp.int32),
            pltpu.VMEM((TCH, CW), jnp.int32),
            pltpu.VMEM((CW,), jnp.int32),
            pltpu.VMEM((CW,), jnp.int32),
            pltpu.VMEM((CW,), jnp.float32),
            pltpu.VMEM_SHARED((_FLATN,), jnp.float32),
            pltpu.SemaphoreType.DMA,
        ],
    )


def _fin_body(accp_ref, hwst_ref, dinvt_ref, b_ref, out_ref):
    t = (accp_ref[0] + accp_ref[1] + hwst_ref[...]) * dinvt_ref[...] + b_ref[...]
    mask = lax.broadcasted_iota(jnp.int32, t.shape, 0) < OUT
    z = jnp.where(mask, t, -jnp.inf)
    m = jnp.max(z, axis=0, keepdims=True)
    ez = jnp.where(mask, jnp.exp(z - m), 0.0)
    lse = jnp.log(jnp.sum(ez, axis=0, keepdims=True)) + m
    out_ref[...] = t - lse


_FBL = 1024


def _tc_fin(accp, hwst, dinvt, b8):
    return pl.pallas_call(
        _fin_body,
        grid=(PADN // _FBL,),
        in_specs=[
            pl.BlockSpec((2, OUTP, _FBL), lambda i: (0, 0, i)),
            pl.BlockSpec((OUTP, _FBL), lambda i: (0, i)),
            pl.BlockSpec((1, _FBL), lambda i: (0, i)),
            pl.BlockSpec((OUTP, 1), lambda i: (0, 0)),
        ],
        out_specs=pl.BlockSpec((OUTP, _FBL), lambda i: (0, i)),
        out_shape=jax.ShapeDtypeStruct((OUTP, PADN), jnp.float32),
    )(accp, hwst, dinvt, b8)


def kernel(x, edge_index, W, b):
    h = _embed_like(x)

    pad = jnp.full((EPAD - E,), N, dtype=jnp.int32)
    src3 = jnp.concatenate([edge_index[0], pad]).reshape(NT, TCH, CW)
    dst3 = jnp.concatenate([edge_index[1], pad]).reshape(NT, TCH, CW)

    zeros1 = jnp.zeros((PADN,), jnp.float32)
    ones128 = jnp.ones((CW,), jnp.float32)
    zeros8 = jnp.zeros((_FLATN,), jnp.float32)

    degp = _sc_deg()(dst3, zeros1, ones128)
    wp = jnp.pad(W, ((0, 0), (0, OUTP - OUT)))
    hws, dinv = _tc_hw(h, wp, degp.T)
    hwst = jnp.pad(hws, ((0, PADN - N), (0, 0))).T
    dinvt = jnp.pad(dinv, ((0, PADN - N), (0, 0))).T
    accf = _sc_edges()(src3, dst3, hwst.reshape(_FLATN), zeros8)
    b8 = jnp.pad(b, (0, OUTP - OUT)).reshape(OUTP, 1)
    outt = _tc_fin(accf.reshape(NC, OUTP, PADN), hwst, dinvt, b8)
    return outt[:OUT, :N].T


# speedup vs baseline: 5.0961x; 1.1933x over previous
import functools

import jax
import jax.numpy as jnp
import numpy as np
from jax import lax
from jax.experimental import pallas as pl
from jax.experimental.pallas import tpu as pltpu
from jax.experimental.pallas import tpu_sc as plsc

N = 10000
F = 128
FEAT_EMB = 15
VAL_EMB = 1
EMB = FEAT_EMB + VAL_EMB
CHANNELS = F * EMB
OUT = 7
E = 640000

OUTP = 8
PADN = 10240
NC = 2
NS = 16
NT = NC * NS
CW = 128
TCH = 160
EPAD = NT * TCH * CW
SL = PADN // NS

@functools.cache
def _mesh():
    return plsc.VectorSubcoreMesh(
        core_axis_name="c", subcore_axis_name="s", num_cores=NC, num_subcores=NS
    )


def _embed_like(x):
    Xt = x.T
    Xc = Xt - Xt.mean(axis=0, keepdims=True)
    U, S, Vt = jnp.linalg.svd(Xc, full_matrices=False)
    feat_emb = U[:, :FEAT_EMB] * S[:FEAT_EMB]
    vals = x.reshape(N * F, 1)
    vals = jnp.repeat(vals, VAL_EMB, axis=1)
    fe = jnp.tile(feat_emb, (N, 1))
    cat = jnp.concatenate([fe, vals], axis=1)
    node_vec = cat.reshape(N, F * EMB)
    mu = node_vec.mean(axis=0, keepdims=True)
    sd = node_vec.std(axis=0, keepdims=True)
    return (node_vec - mu) / (sd + 1e-12)


def _deg_body(dst_hbm, z_hbm, ones_hbm, deg_out, idx_v, ones_v, deg_sh, sem):
    del sem
    c = lax.axis_index("c")
    s = lax.axis_index("s")
    tid = s * NC + c
    pltpu.sync_copy(z_hbm.at[pl.ds(s * SL, SL)], deg_sh.at[pl.ds(s * SL, SL)])
    pltpu.sync_copy(ones_hbm, ones_v)
    pltpu.sync_copy(dst_hbm.at[tid], idx_v)
    plsc.subcore_barrier()

    def step(j, carry):
        pltpu.sync_copy(ones_v, deg_sh.at[idx_v.at[j]], add=True)
        return carry

    lax.fori_loop(0, TCH, step, 0)
    plsc.subcore_barrier()

    @pl.when(s == 0)
    def _():
        pltpu.sync_copy(deg_sh, deg_out.at[c])


@functools.cache
def _sc_deg():
    return pl.kernel(
        _deg_body,
        out_type=jax.ShapeDtypeStruct((NC, PADN), jnp.float32),
        mesh=_mesh(),
        scratch_types=[
            pltpu.VMEM((TCH, CW), jnp.int32),
            pltpu.VMEM((CW,), jnp.float32),
            pltpu.VMEM_SHARED((PADN,), jnp.float32),
            pltpu.SemaphoreType.DMA,
        ],
    )


def _hw_body(h_ref, wp_ref, degt_ref, hws_ref, dinv_ref):
    deg = degt_ref[:, 0] + degt_ref[:, 1] + 1.0
    dinv = lax.rsqrt(deg)
    hw = jnp.dot(h_ref[...], wp_ref[...], preferred_element_type=jnp.float32)
    hws_ref[...] = hw * dinv[:, None]
    dinv_ref[...] = dinv[:, None]


_BL = 400


def _tc_hw(h, wp, degt):
    return pl.pallas_call(
        _hw_body,
        grid=(N // _BL,),
        in_specs=[
            pl.BlockSpec((_BL, CHANNELS), lambda i: (i, 0)),
            pl.BlockSpec((CHANNELS, OUTP), lambda i: (0, 0)),
            pl.BlockSpec((_BL, 2), lambda i: (i, 0)),
        ],
        out_specs=[
            pl.BlockSpec((_BL, OUTP), lambda i: (i, 0)),
            pl.BlockSpec((_BL, 1), lambda i: (i, 0)),
        ],
        out_shape=[
            jax.ShapeDtypeStruct((N, OUTP), jnp.float32),
            jax.ShapeDtypeStruct((N, 1), jnp.float32),
        ],
    )(h, wp, degt)


_FLATN = OUTP * PADN
_SL8 = _FLATN // NS


_PL = OUTP - 1
_SB = 8
_NSB = TCH // _SB


def _edges_body(gidx_hbm, widx_hbm, hwst_hbm, z8_hbm, acc_out,
                gidx, widx, rows, acc_sh, gsem, asem):
    c = lax.axis_index("c")
    s = lax.axis_index("s")
    tid = s * NC + c
    pltpu.sync_copy(z8_hbm.at[pl.ds(s * _SL8, _SL8)],
                    acc_sh.at[pl.ds(s * _SL8, _SL8)])
    plsc.subcore_barrier()

    def step(jo, carry):
        blk = pl.ds(jo * _SB * _PL, _SB * _PL)
        pltpu.sync_copy(gidx_hbm.at[tid, blk], gidx)
        pltpu.sync_copy(widx_hbm.at[tid, blk], widx)
        for b in range(_SB):
            for p in range(_PL):
                pltpu.make_async_copy(
                    hwst_hbm.at[gidx.at[b * _PL + p]], rows.at[p], gsem
                ).start()
            for p in range(_PL):
                pltpu.make_async_copy(
                    hwst_hbm.at[gidx.at[b * _PL + p]], rows.at[p], gsem
                ).wait()
            for p in range(_PL):
                pltpu.make_async_copy(
                    rows.at[p], acc_sh.at[widx.at[b * _PL + p]], asem
                ).start(add=True)
            for p in range(_PL):
                pltpu.make_async_copy(
                    rows.at[p], acc_sh.at[widx.at[b * _PL + p]], asem
                ).wait()
        return carry

    lax.fori_loop(0, _NSB, step, 0)
    plsc.subcore_barrier()

    @pl.when(s == 0)
    def _():
        pltpu.sync_copy(acc_sh, acc_out.at[c])


@functools.cache
def _sc_edges():
    return pl.kernel(
        _edges_body,
        out_type=jax.ShapeDtypeStruct((NC, _FLATN), jnp.float32),
        mesh=_mesh(),
        scratch_types=[
            pltpu.VMEM((_SB * _PL, CW), jnp.int32),
            pltpu.VMEM((_SB * _PL, CW), jnp.int32),
            pltpu.VMEM((_PL, CW), jnp.float32),
            pltpu.VMEM_SHARED((_FLATN,), jnp.float32),
            pltpu.SemaphoreType.DMA,
            pltpu.SemaphoreType.DMA,
        ],
    )


def _fin_body(accp_ref, hwst_ref, dinvt_ref, b_ref, out_ref):
    t = (accp_ref[0] + accp_ref[1] + hwst_ref[...]) * dinvt_ref[...] + b_ref[...]
    mask = lax.broadcasted_iota(jnp.int32, t.shape, 0) < OUT
    z = jnp.where(mask, t, -jnp.inf)
    m = jnp.max(z, axis=0, keepdims=True)
    ez = jnp.where(mask, jnp.exp(z - m), 0.0)
    lse = jnp.log(jnp.sum(ez, axis=0, keepdims=True)) + m
    out_ref[...] = t - lse


_FBL = 1024


def _tc_fin(accp, hwst, dinvt, b8):
    return pl.pallas_call(
        _fin_body,
        grid=(PADN // _FBL,),
        in_specs=[
            pl.BlockSpec((2, OUTP, _FBL), lambda i: (0, 0, i)),
            pl.BlockSpec((OUTP, _FBL), lambda i: (0, i)),
            pl.BlockSpec((1, _FBL), lambda i: (0, i)),
            pl.BlockSpec((OUTP, 1), lambda i: (0, 0)),
        ],
        out_specs=pl.BlockSpec((OUTP, _FBL), lambda i: (0, i)),
        out_shape=jax.ShapeDtypeStruct((OUTP, PADN), jnp.float32),
    )(accp, hwst, dinvt, b8)


def kernel(x, edge_index, W, b):
    h = _embed_like(x)

    pad = jnp.full((EPAD - E,), N, dtype=jnp.int32)
    src3 = jnp.concatenate([edge_index[0], pad]).reshape(NT, TCH, CW)
    dst3 = jnp.concatenate([edge_index[1], pad]).reshape(NT, TCH, CW)

    zeros1 = jnp.zeros((PADN,), jnp.float32)
    ones128 = jnp.ones((CW,), jnp.float32)
    zeros8 = jnp.zeros((_FLATN,), jnp.float32)

    degp = _sc_deg()(dst3, zeros1, ones128)
    wp = jnp.pad(W, ((0, 0), (0, OUTP - OUT)))
    hws, dinv = _tc_hw(h, wp, degp.T)
    hwst = jnp.pad(hws, ((0, PADN - N), (0, 0))).T
    dinvt = jnp.pad(dinv, ((0, PADN - N), (0, 0))).T
    poff = (jnp.arange(_PL, dtype=jnp.int32) * PADN)[None, None, :, None]
    gidx_all = (src3[:, :, None, :] + poff).reshape(NT, TCH * _PL, CW)
    widx_all = (dst3[:, :, None, :] + poff).reshape(NT, TCH * _PL, CW)
    accf = _sc_edges()(gidx_all, widx_all, hwst.reshape(_FLATN), zeros8)
    b8 = jnp.pad(b, (0, OUTP - OUT)).reshape(OUTP, 1)
    outt = _tc_fin(accf.reshape(NC, OUTP, PADN), hwst, dinvt, b8)
    return outt[:OUT, :N].T


# speedup vs baseline: 5.4045x; 1.0605x over previous
import functools

import jax
import jax.numpy as jnp
import numpy as np
from jax import lax
from jax.experimental import pallas as pl
from jax.experimental.pallas import tpu as pltpu
from jax.experimental.pallas import tpu_sc as plsc

N = 10000
F = 128
FEAT_EMB = 15
VAL_EMB = 1
EMB = FEAT_EMB + VAL_EMB
CHANNELS = F * EMB
OUT = 7
E = 640000

OUTP = 8
PADN = 10240
NC = 2
NS = 16
NT = NC * NS
CW = 128
TCH = 160
EPAD = NT * TCH * CW
SL = PADN // NS

@functools.cache
def _mesh():
    return plsc.VectorSubcoreMesh(
        core_axis_name="c", subcore_axis_name="s", num_cores=NC, num_subcores=NS
    )


def _embed_like(x):
    Xt = x.T
    Xc = Xt - Xt.mean(axis=0, keepdims=True)
    U, S, Vt = jnp.linalg.svd(Xc, full_matrices=False)
    feat_emb = U[:, :FEAT_EMB] * S[:FEAT_EMB]
    vals = x.reshape(N * F, 1)
    vals = jnp.repeat(vals, VAL_EMB, axis=1)
    fe = jnp.tile(feat_emb, (N, 1))
    cat = jnp.concatenate([fe, vals], axis=1)
    node_vec = cat.reshape(N, F * EMB)
    mu = node_vec.mean(axis=0, keepdims=True)
    sd = node_vec.std(axis=0, keepdims=True)
    return (node_vec - mu) / (sd + 1e-12)


def _deg_body(dst_hbm, z_hbm, ones_hbm, deg_out, idx_v, ones_v, deg_sh, sem):
    del sem
    c = lax.axis_index("c")
    s = lax.axis_index("s")
    tid = s * NC + c
    pltpu.sync_copy(z_hbm.at[pl.ds(s * SL, SL)], deg_sh.at[pl.ds(s * SL, SL)])
    pltpu.sync_copy(ones_hbm, ones_v)
    pltpu.sync_copy(dst_hbm.at[tid], idx_v)
    plsc.subcore_barrier()

    def step(j, carry):
        pltpu.sync_copy(ones_v, deg_sh.at[idx_v.at[j]], add=True)
        return carry

    lax.fori_loop(0, TCH, step, 0)
    plsc.subcore_barrier()

    @pl.when(s == 0)
    def _():
        pltpu.sync_copy(deg_sh, deg_out.at[c])


@functools.cache
def _sc_deg():
    return pl.kernel(
        _deg_body,
        out_type=jax.ShapeDtypeStruct((NC, PADN), jnp.float32),
        mesh=_mesh(),
        scratch_types=[
            pltpu.VMEM((TCH, CW), jnp.int32),
            pltpu.VMEM((CW,), jnp.float32),
            pltpu.VMEM_SHARED((PADN,), jnp.float32),
            pltpu.SemaphoreType.DMA,
        ],
    )


def _hw_body(h_ref, wp_ref, degt_ref, hws_ref, dinv_ref):
    deg = degt_ref[:, 0] + degt_ref[:, 1] + 1.0
    dinv = lax.rsqrt(deg)
    hw = jnp.dot(h_ref[...], wp_ref[...], preferred_element_type=jnp.float32)
    hws_ref[...] = hw * dinv[:, None]
    dinv_ref[...] = dinv[:, None]


_BL = 400


def _tc_hw(h, wp, degt):
    return pl.pallas_call(
        _hw_body,
        grid=(N // _BL,),
        in_specs=[
            pl.BlockSpec((_BL, CHANNELS), lambda i: (i, 0)),
            pl.BlockSpec((CHANNELS, OUTP), lambda i: (0, 0)),
            pl.BlockSpec((_BL, 2), lambda i: (i, 0)),
        ],
        out_specs=[
            pl.BlockSpec((_BL, OUTP), lambda i: (i, 0)),
            pl.BlockSpec((_BL, 1), lambda i: (i, 0)),
        ],
        out_shape=[
            jax.ShapeDtypeStruct((N, OUTP), jnp.float32),
            jax.ShapeDtypeStruct((N, 1), jnp.float32),
        ],
    )(h, wp, degt)


_FLATN = OUTP * PADN
_SL8 = _FLATN // NS


_PL = OUTP - 1
_SB = 8
_NSB = TCH // _SB


def _edges_body(gidx_hbm, widx_hbm, hwst_hbm, z8_hbm, acc_out,
                gidx, widx, rows, acc_sh, hws_sh, gsem, asem):
    c = lax.axis_index("c")
    s = lax.axis_index("s")
    tid = s * NC + c
    pltpu.sync_copy(z8_hbm.at[pl.ds(s * _SL8, _SL8)],
                    acc_sh.at[pl.ds(s * _SL8, _SL8)])
    pltpu.sync_copy(hwst_hbm.at[pl.ds(s * _SL8, _SL8)],
                    hws_sh.at[pl.ds(s * _SL8, _SL8)])
    plsc.subcore_barrier()

    def step(jo, carry):
        blk = pl.ds(jo * _SB * _PL, _SB * _PL)
        pltpu.sync_copy(gidx_hbm.at[tid, blk], gidx)
        pltpu.sync_copy(widx_hbm.at[tid, blk], widx)
        for b in range(_SB):
            for p in range(_PL):
                pltpu.make_async_copy(
                    hws_sh.at[gidx.at[b * _PL + p]], rows.at[p], gsem
                ).start()
            for p in range(_PL):
                pltpu.make_async_copy(
                    hws_sh.at[gidx.at[b * _PL + p]], rows.at[p], gsem
                ).wait()
            for p in range(_PL):
                pltpu.make_async_copy(
                    rows.at[p], acc_sh.at[widx.at[b * _PL + p]], asem
                ).start(add=True)
            for p in range(_PL):
                pltpu.make_async_copy(
                    rows.at[p], acc_sh.at[widx.at[b * _PL + p]], asem
                ).wait()
        return carry

    lax.fori_loop(0, _NSB, step, 0)
    plsc.subcore_barrier()

    @pl.when(s == 0)
    def _():
        pltpu.sync_copy(acc_sh, acc_out.at[c])


@functools.cache
def _sc_edges():
    return pl.kernel(
        _edges_body,
        out_type=jax.ShapeDtypeStruct((NC, _FLATN), jnp.float32),
        mesh=_mesh(),
        scratch_types=[
            pltpu.VMEM((_SB * _PL, CW), jnp.int32),
            pltpu.VMEM((_SB * _PL, CW), jnp.int32),
            pltpu.VMEM((_PL, CW), jnp.float32),
            pltpu.VMEM_SHARED((_FLATN,), jnp.float32),
            pltpu.VMEM_SHARED((_FLATN,), jnp.float32),
            pltpu.SemaphoreType.DMA,
            pltpu.SemaphoreType.DMA,
        ],
    )


def _fin_body(accp_ref, hwst_ref, dinvt_ref, b_ref, out_ref):
    t = (accp_ref[0] + accp_ref[1] + hwst_ref[...]) * dinvt_ref[...] + b_ref[...]
    mask = lax.broadcasted_iota(jnp.int32, t.shape, 0) < OUT
    z = jnp.where(mask, t, -jnp.inf)
    m = jnp.max(z, axis=0, keepdims=True)
    ez = jnp.where(mask, jnp.exp(z - m), 0.0)
    lse = jnp.log(jnp.sum(ez, axis=0, keepdims=True)) + m
    out_ref[...] = t - lse


_FBL = 1024


def _tc_fin(accp, hwst, dinvt, b8):
    return pl.pallas_call(
        _fin_body,
        grid=(PADN // _FBL,),
        in_specs=[
            pl.BlockSpec((2, OUTP, _FBL), lambda i: (0, 0, i)),
            pl.BlockSpec((OUTP, _FBL), lambda i: (0, i)),
            pl.BlockSpec((1, _FBL), lambda i: (0, i)),
            pl.BlockSpec((OUTP, 1), lambda i: (0, 0)),
        ],
        out_specs=pl.BlockSpec((OUTP, _FBL), lambda i: (0, i)),
        out_shape=jax.ShapeDtypeStruct((OUTP, PADN), jnp.float32),
    )(accp, hwst, dinvt, b8)


def kernel(x, edge_index, W, b):
    h = _embed_like(x)

    pad = jnp.full((EPAD - E,), N, dtype=jnp.int32)
    src3 = jnp.concatenate([edge_index[0], pad]).reshape(NT, TCH, CW)
    dst3 = jnp.concatenate([edge_index[1], pad]).reshape(NT, TCH, CW)

    zeros1 = jnp.zeros((PADN,), jnp.float32)
    ones128 = jnp.ones((CW,), jnp.float32)
    zeros8 = jnp.zeros((_FLATN,), jnp.float32)

    degp = _sc_deg()(dst3, zeros1, ones128)
    wp = jnp.pad(W, ((0, 0), (0, OUTP - OUT)))
    hws, dinv = _tc_hw(h, wp, degp.T)
    hwst = jnp.pad(hws, ((0, PADN - N), (0, 0))).T
    dinvt = jnp.pad(dinv, ((0, PADN - N), (0, 0))).T
    poff = (jnp.arange(_PL, dtype=jnp.int32) * PADN)[None, None, :, None]
    gidx_all = (src3[:, :, None, :] + poff).reshape(NT, TCH * _PL, CW)
    widx_all = (dst3[:, :, None, :] + poff).reshape(NT, TCH * _PL, CW)
    accf = _sc_edges()(gidx_all, widx_all, hwst.reshape(_FLATN), zeros8)
    b8 = jnp.pad(b, (0, OUTP - OUT)).reshape(OUTP, 1)
    outt = _tc_fin(accf.reshape(NC, OUTP, PADN), hwst, dinvt, b8)
    return outt[:OUT, :N].T


# speedup vs baseline: 5.5118x; 1.0199x over previous
import functools

import jax
import jax.numpy as jnp
import numpy as np
from jax import lax
from jax.experimental import pallas as pl
from jax.experimental.pallas import tpu as pltpu
from jax.experimental.pallas import tpu_sc as plsc

N = 10000
F = 128
FEAT_EMB = 15
VAL_EMB = 1
EMB = FEAT_EMB + VAL_EMB
CHANNELS = F * EMB
OUT = 7
E = 640000

OUTP = 8
PADN = 10240
NC = 2
NS = 16
NT = NC * NS
CW = 128
TCH = 160
EPAD = NT * TCH * CW
SL = PADN // NS

@functools.cache
def _mesh():
    return plsc.VectorSubcoreMesh(
        core_axis_name="c", subcore_axis_name="s", num_cores=NC, num_subcores=NS
    )


def _embed_like(x):
    Xt = x.T
    Xc = Xt - Xt.mean(axis=0, keepdims=True)
    U, S, Vt = jnp.linalg.svd(Xc, full_matrices=False)
    feat_emb = U[:, :FEAT_EMB] * S[:FEAT_EMB]
    vals = x.reshape(N * F, 1)
    vals = jnp.repeat(vals, VAL_EMB, axis=1)
    fe = jnp.tile(feat_emb, (N, 1))
    cat = jnp.concatenate([fe, vals], axis=1)
    node_vec = cat.reshape(N, F * EMB)
    mu = node_vec.mean(axis=0, keepdims=True)
    sd = node_vec.std(axis=0, keepdims=True)
    return (node_vec - mu) / (sd + 1e-12)


def _deg_body(dst_hbm, z_hbm, ones_hbm, deg_out, idx_v, ones_v, deg_sh, sem):
    del sem
    c = lax.axis_index("c")
    s = lax.axis_index("s")
    tid = s * NC + c
    pltpu.sync_copy(z_hbm.at[pl.ds(s * SL, SL)], deg_sh.at[pl.ds(s * SL, SL)])
    pltpu.sync_copy(ones_hbm, ones_v)
    pltpu.sync_copy(dst_hbm.at[tid], idx_v)
    plsc.subcore_barrier()

    def step(j, carry):
        pltpu.sync_copy(ones_v, deg_sh.at[idx_v.at[j]], add=True)
        return carry

    lax.fori_loop(0, TCH, step, 0)
    plsc.subcore_barrier()

    @pl.when(s == 0)
    def _():
        pltpu.sync_copy(deg_sh, deg_out.at[c])


@functools.cache
def _sc_deg():
    return pl.kernel(
        _deg_body,
        out_type=jax.ShapeDtypeStruct((NC, PADN), jnp.float32),
        mesh=_mesh(),
        scratch_types=[
            pltpu.VMEM((TCH, CW), jnp.int32),
            pltpu.VMEM((CW,), jnp.float32),
            pltpu.VMEM_SHARED((PADN,), jnp.float32),
            pltpu.SemaphoreType.DMA,
        ],
    )


def _hw_body(h_ref, wp_ref, degt_ref, hws_ref, dinv_ref):
    deg = degt_ref[:, 0] + degt_ref[:, 1] + 1.0
    dinv = lax.rsqrt(deg)
    hw = jnp.dot(h_ref[...], wp_ref[...], preferred_element_type=jnp.float32)
    hws_ref[...] = hw * dinv[:, None]
    dinv_ref[...] = dinv[:, None]


_BL = 400


def _tc_hw(h, wp, degt):
    return pl.pallas_call(
        _hw_body,
        grid=(N // _BL,),
        in_specs=[
            pl.BlockSpec((_BL, CHANNELS), lambda i: (i, 0)),
            pl.BlockSpec((CHANNELS, OUTP), lambda i: (0, 0)),
            pl.BlockSpec((_BL, 2), lambda i: (i, 0)),
        ],
        out_specs=[
            pl.BlockSpec((_BL, OUTP), lambda i: (i, 0)),
            pl.BlockSpec((_BL, 1), lambda i: (i, 0)),
        ],
        out_shape=[
            jax.ShapeDtypeStruct((N, OUTP), jnp.float32),
            jax.ShapeDtypeStruct((N, 1), jnp.float32),
        ],
    )(h, wp, degt)


_FLATN = OUTP * PADN
_SL8 = _FLATN // NS


_PL = OUTP - 1
_SB = 8
_NSB = TCH // _SB


def _edges_body(gidx_hbm, widx_hbm, hwst_hbm, z8_hbm, acc_out,
                gidx, widx, rows, acc_sh, hws_sh, gsem, asem):
    c = lax.axis_index("c")
    s = lax.axis_index("s")
    tid = s * NC + c
    pltpu.sync_copy(z8_hbm.at[pl.ds(s * _SL8, _SL8)],
                    acc_sh.at[pl.ds(s * _SL8, _SL8)])
    pltpu.sync_copy(hwst_hbm.at[pl.ds(s * _SL8, _SL8)],
                    hws_sh.at[pl.ds(s * _SL8, _SL8)])
    plsc.subcore_barrier()

    def step(jo, carry):
        blk = pl.ds(jo * _SB * _PL, _SB * _PL)
        pltpu.sync_copy(gidx_hbm.at[tid, blk], gidx)
        pltpu.sync_copy(widx_hbm.at[tid, blk], widx)
        for b in range(_SB):
            buf = b % 2
            if b >= 2:
                for p in range(_PL):
                    pltpu.make_async_copy(
                        rows.at[buf, p],
                        acc_sh.at[widx.at[(b - 2) * _PL + p]], asem
                    ).wait()
            for p in range(_PL):
                pltpu.make_async_copy(
                    hws_sh.at[gidx.at[b * _PL + p]], rows.at[buf, p], gsem
                ).start()
            for p in range(_PL):
                pltpu.make_async_copy(
                    hws_sh.at[gidx.at[b * _PL + p]], rows.at[buf, p], gsem
                ).wait()
            for p in range(_PL):
                pltpu.make_async_copy(
                    rows.at[buf, p], acc_sh.at[widx.at[b * _PL + p]], asem
                ).start(add=True)
        for b in range(_SB - 2, _SB):
            for p in range(_PL):
                pltpu.make_async_copy(
                    rows.at[b % 2, p], acc_sh.at[widx.at[b * _PL + p]], asem
                ).wait()
        return carry

    lax.fori_loop(0, _NSB, step, 0)
    plsc.subcore_barrier()

    @pl.when(s == 0)
    def _():
        pltpu.sync_copy(acc_sh, acc_out.at[c])


@functools.cache
def _sc_edges():
    return pl.kernel(
        _edges_body,
        out_type=jax.ShapeDtypeStruct((NC, _FLATN), jnp.float32),
        mesh=_mesh(),
        scratch_types=[
            pltpu.VMEM((_SB * _PL, CW), jnp.int32),
            pltpu.VMEM((_SB * _PL, CW), jnp.int32),
            pltpu.VMEM((2, _PL, CW), jnp.float32),
            pltpu.VMEM_SHARED((_FLATN,), jnp.float32),
            pltpu.VMEM_SHARED((_FLATN,), jnp.float32),
            pltpu.SemaphoreType.DMA,
            pltpu.SemaphoreType.DMA,
        ],
    )


def _fin_body(accp_ref, hwst_ref, dinvt_ref, b_ref, out_ref):
    t = (accp_ref[0] + accp_ref[1] + hwst_ref[...]) * dinvt_ref[...] + b_ref[...]
    mask = lax.broadcasted_iota(jnp.int32, t.shape, 0) < OUT
    z = jnp.where(mask, t, -jnp.inf)
    m = jnp.max(z, axis=0, keepdims=True)
    ez = jnp.where(mask, jnp.exp(z - m), 0.0)
    lse = jnp.log(jnp.sum(ez, axis=0, keepdims=True)) + m
    out_ref[...] = t - lse


_FBL = 1024


def _tc_fin(accp, hwst, dinvt, b8):
    return pl.pallas_call(
        _fin_body,
        grid=(PADN // _FBL,),
        in_specs=[
            pl.BlockSpec((2, OUTP, _FBL), lambda i: (0, 0, i)),
            pl.BlockSpec((OUTP, _FBL), lambda i: (0, i)),
            pl.BlockSpec((1, _FBL), lambda i: (0, i)),
            pl.BlockSpec((OUTP, 1), lambda i: (0, 0)),
        ],
        out_specs=pl.BlockSpec((OUTP, _FBL), lambda i: (0, i)),
        out_shape=jax.ShapeDtypeStruct((OUTP, PADN), jnp.float32),
    )(accp, hwst, dinvt, b8)


def kernel(x, edge_index, W, b):
    h = _embed_like(x)

    pad = jnp.full((EPAD - E,), N, dtype=jnp.int32)
    src3 = jnp.concatenate([edge_index[0], pad]).reshape(NT, TCH, CW)
    dst3 = jnp.concatenate([edge_index[1], pad]).reshape(NT, TCH, CW)

    zeros1 = jnp.zeros((PADN,), jnp.float32)
    ones128 = jnp.ones((CW,), jnp.float32)
    zeros8 = jnp.zeros((_FLATN,), jnp.float32)

    degp = _sc_deg()(dst3, zeros1, ones128)
    wp = jnp.pad(W, ((0, 0), (0, OUTP - OUT)))
    hws, dinv = _tc_hw(h, wp, degp.T)
    hwst = jnp.pad(hws, ((0, PADN - N), (0, 0))).T
    dinvt = jnp.pad(dinv, ((0, PADN - N), (0, 0))).T
    poff = (jnp.arange(_PL, dtype=jnp.int32) * PADN)[None, None, :, None]
    gidx_all = (src3[:, :, None, :] + poff).reshape(NT, TCH * _PL, CW)
    widx_all = (dst3[:, :, None, :] + poff).reshape(NT, TCH * _PL, CW)
    accf = _sc_edges()(gidx_all, widx_all, hwst.reshape(_FLATN), zeros8)
    b8 = jnp.pad(b, (0, OUTP - OUT)).reshape(OUTP, 1)
    outt = _tc_fin(accf.reshape(NC, OUTP, PADN), hwst, dinvt, b8)
    return outt[:OUT, :N].T


# speedup vs baseline: 5.5548x; 1.0078x over previous
import functools

import jax
import jax.numpy as jnp
import numpy as np
from jax import lax
from jax.experimental import pallas as pl
from jax.experimental.pallas import tpu as pltpu
from jax.experimental.pallas import tpu_sc as plsc

N = 10000
F = 128
FEAT_EMB = 15
VAL_EMB = 1
EMB = FEAT_EMB + VAL_EMB
CHANNELS = F * EMB
OUT = 7
E = 640000

OUTP = 8
PADN = 10240
NC = 2
NS = 16
NT = NC * NS
CW = 128
TCH = 160
EPAD = NT * TCH * CW
SL = PADN // NS

@functools.cache
def _mesh():
    return plsc.VectorSubcoreMesh(
        core_axis_name="c", subcore_axis_name="s", num_cores=NC, num_subcores=NS
    )


def _embed_like(x):
    Xt = x.T
    Xc = Xt - Xt.mean(axis=0, keepdims=True)
    U, S, Vt = jnp.linalg.svd(Xc, full_matrices=False)
    feat_emb = U[:, :FEAT_EMB] * S[:FEAT_EMB]
    vals = x.reshape(N * F, 1)
    vals = jnp.repeat(vals, VAL_EMB, axis=1)
    fe = jnp.tile(feat_emb, (N, 1))
    cat = jnp.concatenate([fe, vals], axis=1)
    node_vec = cat.reshape(N, F * EMB)
    mu = node_vec.mean(axis=0, keepdims=True)
    sd = node_vec.std(axis=0, keepdims=True)
    return (node_vec - mu) / (sd + 1e-12)


def _deg_body(dst_hbm, z_hbm, ones_hbm, deg_out, idx_v, ones_v, deg_sh, sem):
    del sem
    c = lax.axis_index("c")
    s = lax.axis_index("s")
    tid = s * NC + c
    pltpu.sync_copy(z_hbm.at[pl.ds(s * SL, SL)], deg_sh.at[pl.ds(s * SL, SL)])
    pltpu.sync_copy(ones_hbm, ones_v)
    pltpu.sync_copy(dst_hbm.at[tid], idx_v)
    plsc.subcore_barrier()

    def step(j, carry):
        pltpu.sync_copy(ones_v, deg_sh.at[idx_v.at[j]], add=True)
        return carry

    lax.fori_loop(0, TCH, step, 0)
    plsc.subcore_barrier()

    @pl.when(s == 0)
    def _():
        pltpu.sync_copy(deg_sh, deg_out.at[c])


@functools.cache
def _sc_deg():
    return pl.kernel(
        _deg_body,
        out_type=jax.ShapeDtypeStruct((NC, PADN), jnp.float32),
        mesh=_mesh(),
        scratch_types=[
            pltpu.VMEM((TCH, CW), jnp.int32),
            pltpu.VMEM((CW,), jnp.float32),
            pltpu.VMEM_SHARED((PADN,), jnp.float32),
            pltpu.SemaphoreType.DMA,
        ],
    )


def _hw_body(h_ref, wp_ref, degt_ref, hws_ref, dinv_ref):
    deg = degt_ref[:, 0] + degt_ref[:, 1] + 1.0
    dinv = lax.rsqrt(deg)
    hw = jnp.dot(h_ref[...], wp_ref[...], preferred_element_type=jnp.float32)
    hws_ref[...] = hw * dinv[:, None]
    dinv_ref[...] = dinv[:, None]


_BL = 400


def _tc_hw(h, wp, degt):
    return pl.pallas_call(
        _hw_body,
        grid=(N // _BL,),
        in_specs=[
            pl.BlockSpec((_BL, CHANNELS), lambda i: (i, 0)),
            pl.BlockSpec((CHANNELS, OUTP), lambda i: (0, 0)),
            pl.BlockSpec((_BL, 2), lambda i: (i, 0)),
        ],
        out_specs=[
            pl.BlockSpec((_BL, OUTP), lambda i: (i, 0)),
            pl.BlockSpec((_BL, 1), lambda i: (i, 0)),
        ],
        out_shape=[
            jax.ShapeDtypeStruct((N, OUTP), jnp.float32),
            jax.ShapeDtypeStruct((N, 1), jnp.float32),
        ],
    )(h, wp, degt)


_FLATN = OUTP * PADN
_SL8 = _FLATN // NS


_PL = OUTP - 1
_SB = 8
_NSB = TCH // _SB


def _edges_body(gidx_hbm, widx_hbm, hwst_hbm, z8_hbm, acc_out,
                gidx, widx, rows, acc_sh, hws_sh, gsem, asem, isem):
    c = lax.axis_index("c")
    s = lax.axis_index("s")
    tid = s * NC + c
    pltpu.sync_copy(z8_hbm.at[pl.ds(s * _SL8, _SL8)],
                    acc_sh.at[pl.ds(s * _SL8, _SL8)])
    pltpu.sync_copy(hwst_hbm.at[pl.ds(s * _SL8, _SL8)],
                    hws_sh.at[pl.ds(s * _SL8, _SL8)])
    plsc.subcore_barrier()

    def stage_start(jo, buf):
        blk = pl.ds(jo * _SB * _PL, _SB * _PL)
        pltpu.make_async_copy(gidx_hbm.at[tid, blk], gidx.at[buf], isem).start()
        pltpu.make_async_copy(widx_hbm.at[tid, blk], widx.at[buf], isem).start()

    def stage_wait(jo, buf):
        blk = pl.ds(jo * _SB * _PL, _SB * _PL)
        pltpu.make_async_copy(gidx_hbm.at[tid, blk], gidx.at[buf], isem).wait()
        pltpu.make_async_copy(widx_hbm.at[tid, blk], widx.at[buf], isem).wait()

    stage_start(0, 0)

    def step(jo, carry):
        ib = lax.rem(jo, 2)
        stage_wait(jo, ib)

        @pl.when(jo + 1 < _NSB)
        def _():
            stage_start(jo + 1, 1 - ib)

        gidx_b = gidx.at[ib]
        widx_b = widx.at[ib]
        for b in range(_SB):
            buf = b % 2
            if b >= 2:
                for p in range(_PL):
                    pltpu.make_async_copy(
                        rows.at[buf, p],
                        acc_sh.at[widx_b.at[(b - 2) * _PL + p]], asem
                    ).wait()
            for p in range(_PL):
                pltpu.make_async_copy(
                    hws_sh.at[gidx_b.at[b * _PL + p]], rows.at[buf, p], gsem
                ).start()
            for p in range(_PL):
                pltpu.make_async_copy(
                    hws_sh.at[gidx_b.at[b * _PL + p]], rows.at[buf, p], gsem
                ).wait()
            for p in range(_PL):
                pltpu.make_async_copy(
                    rows.at[buf, p], acc_sh.at[widx_b.at[b * _PL + p]], asem
                ).start(add=True)
        for b in range(_SB - 2, _SB):
            for p in range(_PL):
                pltpu.make_async_copy(
                    rows.at[b % 2, p], acc_sh.at[widx_b.at[b * _PL + p]], asem
                ).wait()
        return carry

    lax.fori_loop(0, _NSB, step, 0)
    plsc.subcore_barrier()

    @pl.when(s == 0)
    def _():
        pltpu.sync_copy(acc_sh, acc_out.at[c])


@functools.cache
def _sc_edges():
    return pl.kernel(
        _edges_body,
        out_type=jax.ShapeDtypeStruct((NC, _FLATN), jnp.float32),
        mesh=_mesh(),
        scratch_types=[
            pltpu.VMEM((2, _SB * _PL, CW), jnp.int32),
            pltpu.VMEM((2, _SB * _PL, CW), jnp.int32),
            pltpu.VMEM((2, _PL, CW), jnp.float32),
            pltpu.VMEM_SHARED((_FLATN,), jnp.float32),
            pltpu.VMEM_SHARED((_FLATN,), jnp.float32),
            pltpu.SemaphoreType.DMA,
            pltpu.SemaphoreType.DMA,
            pltpu.SemaphoreType.DMA,
        ],
    )


def _fin_body(accp_ref, hwst_ref, dinvt_ref, b_ref, out_ref):
    t = (accp_ref[0] + accp_ref[1] + hwst_ref[...]) * dinvt_ref[...] + b_ref[...]
    mask = lax.broadcasted_iota(jnp.int32, t.shape, 0) < OUT
    z = jnp.where(mask, t, -jnp.inf)
    m = jnp.max(z, axis=0, keepdims=True)
    ez = jnp.where(mask, jnp.exp(z - m), 0.0)
    lse = jnp.log(jnp.sum(ez, axis=0, keepdims=True)) + m
    out_ref[...] = t - lse


_FBL = 1024


def _tc_fin(accp, hwst, dinvt, b8):
    return pl.pallas_call(
        _fin_body,
        grid=(PADN // _FBL,),
        in_specs=[
            pl.BlockSpec((2, OUTP, _FBL), lambda i: (0, 0, i)),
            pl.BlockSpec((OUTP, _FBL), lambda i: (0, i)),
            pl.BlockSpec((1, _FBL), lambda i: (0, i)),
            pl.BlockSpec((OUTP, 1), lambda i: (0, 0)),
        ],
        out_specs=pl.BlockSpec((OUTP, _FBL), lambda i: (0, i)),
        out_shape=jax.ShapeDtypeStruct((OUTP, PADN), jnp.float32),
    )(accp, hwst, dinvt, b8)


def kernel(x, edge_index, W, b):
    h = _embed_like(x)

    pad = jnp.full((EPAD - E,), N, dtype=jnp.int32)
    src3 = jnp.concatenate([edge_index[0], pad]).reshape(NT, TCH, CW)
    dst3 = jnp.concatenate([edge_index[1], pad]).reshape(NT, TCH, CW)

    zeros1 = jnp.zeros((PADN,), jnp.float32)
    ones128 = jnp.ones((CW,), jnp.float32)
    zeros8 = jnp.zeros((_FLATN,), jnp.float32)

    degp = _sc_deg()(dst3, zeros1, ones128)
    wp = jnp.pad(W, ((0, 0), (0, OUTP - OUT)))
    hws, dinv = _tc_hw(h, wp, degp.T)
    hwst = jnp.pad(hws, ((0, PADN - N), (0, 0))).T
    dinvt = jnp.pad(dinv, ((0, PADN - N), (0, 0))).T
    poff = (jnp.arange(_PL, dtype=jnp.int32) * PADN)[None, None, :, None]
    gidx_all = (src3[:, :, None, :] + poff).reshape(NT, TCH * _PL, CW)
    widx_all = (dst3[:, :, None, :] + poff).reshape(NT, TCH * _PL, CW)
    accf = _sc_edges()(gidx_all, widx_all, hwst.reshape(_FLATN), zeros8)
    b8 = jnp.pad(b, (0, OUTP - OUT)).reshape(OUTP, 1)
    outt = _tc_fin(accf.reshape(NC, OUTP, PADN), hwst, dinvt, b8)
    return outt[:OUT, :N].T
